# Initial kernel scaffold; baseline (speedup 1.0000x reference)
import jax
import jax.numpy as jnp
from jax.experimental import pallas as pl


def kernel(h, edge_index, W, b):
    raise NotImplementedError("write your pallas kernel here")


# speedup vs baseline: 6.2698x; 6.2698x over previous
import functools

import jax
import jax.numpy as jnp
from jax import lax
from jax.experimental import pallas as pl
from jax.experimental.pallas import tpu as pltpu
from jax.experimental.pallas import tpu_sc as plsc

N = 10000
E = 320000
D = 128
NC, NS, L = 2, 16, 16
NW = NC * NS
EPT = E // NW
K = 80
NCHUNK = EPT // K
NPAD = 10240
RPT = NPAD // NS
RB = 128
NCOPY = RPT // RB
DPT = 640
DPAD = DPT * NS


def _sc_segment_sum(h, src, dst):
  mesh = plsc.VectorSubcoreMesh(core_axis_name="c", subcore_axis_name="s")

  @functools.partial(
      pl.kernel,
      out_type=[
          jax.ShapeDtypeStruct((NC, NPAD, D), jnp.float32),
          jax.ShapeDtypeStruct((NC * DPAD,), jnp.float32),
      ],
      mesh=mesh,
      scratch_types=[
          pltpu.VMEM_SHARED((NPAD, D), jnp.float32),
          pltpu.VMEM_SHARED((DPAD,), jnp.float32),
          pltpu.VMEM((RB, D), jnp.float32),
          pltpu.VMEM((DPT,), jnp.float32),
          pltpu.VMEM((K,), jnp.int32),
          pltpu.VMEM((K,), jnp.int32),
          pltpu.VMEM((K, D), jnp.float32),
          pltpu.VMEM((K,), jnp.float32),
          pltpu.SemaphoreType.DMA,
      ],
  )
  def seg_sum(h_hbm, src_hbm, dst_hbm, psum_hbm, dcnt_hbm,
              accum, dega, zbuf, dbuf, isrc, idst, rows, ones, sem):
    c = lax.axis_index("c")
    s = lax.axis_index("s")
    wid = c * NS + s

    zero16 = jnp.zeros((L,), jnp.float32)
    one16 = jnp.ones((L,), jnp.float32)

    def zrow(i, carry):
      for j in range(D // L):
        zbuf[i, pl.ds(j * L, L)] = zero16
      return carry

    lax.fori_loop(0, RB, zrow, 0)

    def zdeg(i, carry):
      dbuf[pl.ds(i * L, L)] = zero16
      return carry

    lax.fori_loop(0, DPT // L, zdeg, 0)

    for j in range(K // L):
      ones[pl.ds(j * L, L)] = one16

    for kk in range(NCOPY):
      pltpu.sync_copy(zbuf, accum.at[pl.ds(s * RPT + kk * RB, RB)])
    pltpu.sync_copy(dbuf, dega.at[pl.ds(s * DPT, DPT)])
    plsc.subcore_barrier()

    ebase = wid * EPT

    def body(i, carry):
      off = ebase + i * K
      pltpu.sync_copy(src_hbm.at[pl.ds(off, K)], isrc)
      pltpu.sync_copy(dst_hbm.at[pl.ds(off, K)], idst)
      pltpu.async_copy(h_hbm.at[isrc], rows, sem).wait()
      pltpu.sync_copy(rows, accum.at[idst], add=True)
      pltpu.sync_copy(ones, dega.at[idst], add=True)
      return carry

    lax.fori_loop(0, NCHUNK, body, 0)
    plsc.subcore_barrier()

    for kk in range(NCOPY):
      r0 = s * RPT + kk * RB
      pltpu.sync_copy(accum.at[pl.ds(r0, RB)], zbuf)
      pltpu.sync_copy(zbuf, psum_hbm.at[c, pl.ds(r0, RB)])
    pltpu.sync_copy(dega.at[pl.ds(s * DPT, DPT)], dbuf)
    pltpu.sync_copy(dbuf, dcnt_hbm.at[pl.ds(c * DPAD + s * DPT, DPT)])

  return seg_sum(h, src, dst)


def _dense(h, psum, dT, W1, W2, b2):
  G = 10
  BR = N // G

  def body(h_ref, p_ref, d_ref, w1_ref, w2_ref, b_ref, o_ref):
    hh = h_ref[...]
    ssum = p_ref[0] + p_ref[1]
    deg = d_ref[:, 0:1] + d_ref[:, 1:2]
    cc = ssum / jnp.maximum(deg, 1.0)
    z = (jnp.dot(hh, w1_ref[...], preferred_element_type=jnp.float32)
         + jnp.dot(cc, w2_ref[...], preferred_element_type=jnp.float32)
         + b_ref[...])
    nrm = jnp.sqrt(jnp.sum(z * z, axis=1, keepdims=True))
    o_ref[...] = hh + z / jnp.maximum(nrm, 1e-12)

  return pl.pallas_call(
      body,
      grid=(G,),
      in_specs=[
          pl.BlockSpec((BR, D), lambda i: (i, 0)),
          pl.BlockSpec((NC, BR, D), lambda i: (0, i, 0)),
          pl.BlockSpec((BR, 2), lambda i: (i, 0)),
          pl.BlockSpec((D, D), lambda i: (0, 0)),
          pl.BlockSpec((D, D), lambda i: (0, 0)),
          pl.BlockSpec((1, D), lambda i: (0, 0)),
      ],
      out_specs=pl.BlockSpec((BR, D), lambda i: (i, 0)),
      out_shape=jax.ShapeDtypeStruct((N, D), jnp.float32),
  )(h, psum, dT, W1, W2, b2)


def kernel(h, edge_index, W, b):
  ei = edge_index.astype(jnp.int32)
  src = ei[0]
  dst = ei[1]
  psum, dcnt = _sc_segment_sum(h, src, dst)
  dT = jnp.transpose(dcnt.reshape(NC, DPAD)[:, :N])
  W1 = W[:D]
  W2 = W[D:]
  b2 = b.reshape(1, D)
  return _dense(h, psum, dT, W1, W2, b2)
